# baseline (speedup 1.0000x reference)
import functools

import jax
import jax.numpy as jnp
from jax import lax
from jax.experimental import pallas as pl
from jax.experimental.pallas import tpu as pltpu
from jax.experimental.pallas import tpu_sc as plsc

_EPS = 0.5
_NC = 2
_NS = 16
_L = 16
_GPC = 4


def _make_sc_call(B, N):
    nw = _NC * _NS
    rows_per_w = B // nw
    groups = rows_per_w // _L
    nchunks = groups // _GPC
    chunk_words = _GPC * _L * N
    halves = []
    h = 1
    while h < N:
        halves.append(h)
        h *= 2
    halves.reverse()

    mesh = plsc.VectorSubcoreMesh(
        core_axis_name="c", subcore_axis_name="s",
        num_cores=_NC, num_subcores=_NS)

    @functools.partial(
        pl.kernel,
        out_type=jax.ShapeDtypeStruct((B,), jnp.float32),
        mesh=mesh,
        compiler_params=pltpu.CompilerParams(needs_layout_passes=False),
        scratch_types=[
            pltpu.VMEM((chunk_words,), jnp.float32),
            pltpu.VMEM((chunk_words,), jnp.float32),
            pltpu.VMEM((N * _L,), jnp.float32),
            pltpu.VMEM((rows_per_w,), jnp.float32),
            pltpu.VMEM((rows_per_w,), jnp.float32),
            pltpu.SemaphoreType.DMA,
            pltpu.SemaphoreType.DMA,
        ],
    )
    def sc_call(theta_hbm, t_hbm, out_hbm, th_a, th_b, cum_v, t_v, x_v,
                sem0, sem1):
        bufs = (th_a, th_b)
        sems = (sem0, sem1)
        wid = lax.axis_index("s") * _NC + lax.axis_index("c")
        row0 = wid * rows_per_w
        pltpu.sync_copy(t_hbm.at[pl.ds(row0, rows_per_w)], t_v)
        lane = lax.iota(jnp.int32, _L)
        lane_row_off = lane * N

        def chunk_src(ci):
            return theta_hbm.at[pl.ds((row0 + ci * _GPC * _L) * N,
                                      chunk_words)]

        pltpu.async_copy(chunk_src(0), th_a, sem0)

        def process_group(buf, u, g):
            fidx0 = lane_row_off + u * (_L * N)
            zf = jnp.zeros((_L,), jnp.float32)

            def pass1(m, carry):
                c, fidx = carry
                for uu in range(_L):
                    th = plsc.load_gather(buf, [fidx])
                    c = c + th
                    cum_v[pl.ds(m * (_L * _L) + uu * _L, _L)] = c
                    fidx = fidx + 1
                return c, fidx

            lax.fori_loop(0, N // _L, pass1, (zf, fidx0))

            tv = t_v[pl.ds(g * _L, _L)]
            pos = jnp.zeros((_L,), jnp.int32)
            for half in halves:
                probe = (pos + (half - 1)) * _L + lane
                v = plsc.load_gather(cum_v, [probe])
                pos = jnp.where(v < tv, pos + half, pos)
            v = plsc.load_gather(cum_v, [pos * _L + lane])
            idx = pos + jnp.where(v < tv, 1, 0)
            idxc = jnp.minimum(idx, N - 1)
            j = jnp.maximum(idxc - 1, 0)
            cs_j = plsc.load_gather(cum_v, [j * _L + lane])
            th_next = plsc.load_gather(buf, [fidx0 + idxc])
            s1 = (tv - cs_j) / th_next
            jnf = idxc.astype(jnp.float32)
            jf = j.astype(jnp.float32)
            x_cand = jnf - _EPS + 2.0 * _EPS * s1
            x = jnp.where(jnp.logical_and(s1 == 0.0, j > 0),
                          jf - 1.0 + _EPS, x_cand)
            x_v[pl.ds(g * _L, _L)] = x

        def chunkpair(cp, _):
            for par in range(2):
                ci = 2 * cp + par
                pltpu.make_async_copy(chunk_src(0), bufs[par],
                                      sems[par]).wait()
                nci = ci + 1

                @pl.when(nci < nchunks)
                def _():
                    pltpu.async_copy(chunk_src(nci), bufs[1 - par],
                                     sems[1 - par])

                for u in range(_GPC):
                    process_group(bufs[par], u, ci * _GPC + u)
            return 0

        lax.fori_loop(0, nchunks // 2, chunkpair, 0)
        pltpu.sync_copy(x_v, out_hbm.at[pl.ds(row0, rows_per_w)])

    return sc_call


@jax.jit
def kernel(theta, t, Y_train):
    B, N = theta.shape
    del Y_train
    return _make_sc_call(B, N)(theta.reshape(-1), t)


# speedup vs baseline: 1.3690x; 1.3690x over previous
import functools

import jax
import jax.numpy as jnp
from jax import lax
from jax.experimental import pallas as pl
from jax.experimental.pallas import tpu as pltpu
from jax.experimental.pallas import tpu_sc as plsc

_EPS = 0.5
_NC = 2
_NS = 16
_L = 16
_GPC = 4


def _make_sc_call(B, N):
    nw = _NC * _NS
    rows_per_w = B // nw
    groups = rows_per_w // _L
    nchunks = groups // _GPC
    chunk_words = _GPC * _L * N
    halves = []
    h = 1
    while h < N:
        halves.append(h)
        h *= 2
    halves.reverse()

    mesh = plsc.VectorSubcoreMesh(
        core_axis_name="c", subcore_axis_name="s",
        num_cores=_NC, num_subcores=_NS)

    @functools.partial(
        pl.kernel,
        out_type=jax.ShapeDtypeStruct((B,), jnp.float32),
        mesh=mesh,
        compiler_params=pltpu.CompilerParams(needs_layout_passes=False),
        scratch_types=[
            pltpu.VMEM((chunk_words,), jnp.float32),
            pltpu.VMEM((chunk_words,), jnp.float32),
            pltpu.VMEM((_GPC * N * _L,), jnp.float32),
            pltpu.VMEM((rows_per_w,), jnp.float32),
            pltpu.VMEM((rows_per_w,), jnp.float32),
            pltpu.SemaphoreType.DMA,
            pltpu.SemaphoreType.DMA,
        ],
    )
    def sc_call(theta_hbm, t_hbm, out_hbm, th_a, th_b, cum_v, t_v, x_v,
                sem0, sem1):
        bufs = (th_a, th_b)
        sems = (sem0, sem1)
        wid = lax.axis_index("s") * _NC + lax.axis_index("c")
        row0 = wid * rows_per_w
        pltpu.sync_copy(t_hbm.at[pl.ds(row0, rows_per_w)], t_v)
        lane = lax.iota(jnp.int32, _L)
        lane_row_off = lane * N

        def chunk_src(ci):
            return theta_hbm.at[pl.ds((row0 + ci * _GPC * _L) * N,
                                      chunk_words)]

        pltpu.async_copy(chunk_src(0), th_a, sem0)

        def process_chunk(buf, ci):
            unroll = 8
            fidx0 = [lane_row_off + u * (_L * N) for u in range(_GPC)]
            zf = jnp.zeros((_L,), jnp.float32)

            def pass1(m, carry):
                cs = list(carry[:_GPC])
                fs = list(carry[_GPC:2 * _GPC])
                ths = list(carry[2 * _GPC:])
                for uu in range(unroll):
                    nfs = [f + 1 for f in fs]
                    nths = [plsc.load_gather(buf, [nfs[u]])
                            for u in range(_GPC)]
                    for u in range(_GPC):
                        cs[u] = cs[u] + ths[u]
                        cum_v[pl.ds(u * (N * _L) + m * (unroll * _L)
                                    + uu * _L, _L)] = cs[u]
                    fs, ths = nfs, nths
                return tuple(cs) + tuple(fs) + tuple(ths)

            th0 = [plsc.load_gather(buf, [fidx0[u]]) for u in range(_GPC)]
            lax.fori_loop(0, N // unroll, pass1,
                          (zf,) * _GPC + tuple(fidx0) + tuple(th0))

            tvs = [t_v[pl.ds((ci * _GPC + u) * _L, _L)] for u in range(_GPC)]
            poss = [jnp.zeros((_L,), jnp.int32) for _ in range(_GPC)]
            cbase = [u * (N * _L) for u in range(_GPC)]
            for half in halves:
                vs = [plsc.load_gather(
                    cum_v, [cbase[u] + (poss[u] + (half - 1)) * _L + lane])
                    for u in range(_GPC)]
                poss = [jnp.where(vs[u] < tvs[u], poss[u] + half, poss[u])
                        for u in range(_GPC)]
            for u in range(_GPC):
                tv, pos = tvs[u], poss[u]
                v = plsc.load_gather(cum_v, [cbase[u] + pos * _L + lane])
                idx = pos + jnp.where(v < tv, 1, 0)
                idxc = jnp.minimum(idx, N - 1)
                j = jnp.maximum(idxc - 1, 0)
                cs_j = plsc.load_gather(cum_v, [cbase[u] + j * _L + lane])
                th_next = plsc.load_gather(buf, [fidx0[u] + idxc])
                s1 = (tv - cs_j) / th_next
                jnf = idxc.astype(jnp.float32)
                jf = j.astype(jnp.float32)
                x_cand = jnf - _EPS + 2.0 * _EPS * s1
                x = jnp.where(jnp.logical_and(s1 == 0.0, j > 0),
                              jf - 1.0 + _EPS, x_cand)
                x_v[pl.ds((ci * _GPC + u) * _L, _L)] = x

        def chunkpair(cp, _):
            for par in range(2):
                ci = 2 * cp + par
                pltpu.make_async_copy(chunk_src(0), bufs[par],
                                      sems[par]).wait()
                nci = ci + 1

                @pl.when(nci < nchunks)
                def _():
                    pltpu.async_copy(chunk_src(nci), bufs[1 - par],
                                     sems[1 - par])

                process_chunk(bufs[par], ci)
            return 0

        lax.fori_loop(0, nchunks // 2, chunkpair, 0)
        pltpu.sync_copy(x_v, out_hbm.at[pl.ds(row0, rows_per_w)])

    return sc_call


@jax.jit
def kernel(theta, t, Y_train):
    B, N = theta.shape
    del Y_train
    return _make_sc_call(B, N)(theta.reshape(-1), t)


# speedup vs baseline: 1.5833x; 1.1565x over previous
import functools

import jax
import jax.numpy as jnp
from jax import lax
from jax.experimental import pallas as pl
from jax.experimental.pallas import tpu as pltpu
from jax.experimental.pallas import tpu_sc as plsc

_EPS = 0.5
_NC = 2
_NS = 16
_L = 16
_GPC = 4


def _make_sc_call(B, N):
    nw = _NC * _NS
    rows_per_w = B // nw
    groups = rows_per_w // _L
    nchunks = groups // _GPC
    chunk_rows = _GPC * _L
    rstride = N + 1
    halves = []
    h = 1
    while h < N:
        halves.append(h)
        h *= 2
    halves.reverse()

    mesh = plsc.VectorSubcoreMesh(
        core_axis_name="c", subcore_axis_name="s",
        num_cores=_NC, num_subcores=_NS)

    @functools.partial(
        pl.kernel,
        out_type=jax.ShapeDtypeStruct((B,), jnp.float32),
        mesh=mesh,
        compiler_params=pltpu.CompilerParams(needs_layout_passes=False),
        scratch_types=[
            pltpu.VMEM((chunk_rows, rstride), jnp.float32),
            pltpu.VMEM((chunk_rows, rstride), jnp.float32),
            pltpu.VMEM((_GPC * N * _L,), jnp.float32),
            pltpu.VMEM((rows_per_w,), jnp.float32),
            pltpu.VMEM((rows_per_w,), jnp.float32),
            pltpu.SemaphoreType.DMA,
            pltpu.SemaphoreType.DMA,
        ],
    )
    def sc_call(theta_hbm, t_hbm, out_hbm, th_a, th_b, cum_v, t_v, x_v,
                sem0, sem1):
        bufs = (th_a, th_b)
        sems = (sem0, sem1)
        wid = lax.axis_index("s") * _NC + lax.axis_index("c")
        row0 = wid * rows_per_w
        pltpu.sync_copy(t_hbm.at[pl.ds(row0, rows_per_w)], t_v)
        lane = lax.iota(jnp.int32, _L)

        def chunk_src(ci):
            return theta_hbm.at[pl.ds(row0 + ci * chunk_rows, chunk_rows), :]

        def chunk_dst(buf):
            return buf.at[:, pl.ds(0, N)]

        pltpu.async_copy(chunk_src(0), chunk_dst(th_a), sem0)

        def process_chunk(buf, ci):
            unroll = 8
            rows = [u * _L + lane for u in range(_GPC)]
            zf = jnp.zeros((_L,), jnp.float32)

            def pass1(m, carry):
                cs = list(carry[:_GPC])
                col = carry[_GPC]
                ths = list(carry[_GPC + 1:])
                for uu in range(unroll):
                    ncol = col + 1
                    nths = [plsc.load_gather(buf, [rows[u], ncol])
                            for u in range(_GPC)]
                    for u in range(_GPC):
                        cs[u] = cs[u] + ths[u]
                        cum_v[pl.ds(u * (N * _L) + m * (unroll * _L)
                                    + uu * _L, _L)] = cs[u]
                    col, ths = ncol, nths
                return tuple(cs) + (col,) + tuple(ths)

            col0 = jnp.zeros((_L,), jnp.int32)
            th0 = [plsc.load_gather(buf, [rows[u], col0])
                   for u in range(_GPC)]
            lax.fori_loop(0, N // unroll, pass1,
                          (zf,) * _GPC + (col0,) + tuple(th0))

            tvs = [t_v[pl.ds((ci * _GPC + u) * _L, _L)] for u in range(_GPC)]
            poss = [jnp.zeros((_L,), jnp.int32) for _ in range(_GPC)]
            cbase = [u * (N * _L) for u in range(_GPC)]
            for half in halves:
                vs = [plsc.load_gather(
                    cum_v, [cbase[u] + (poss[u] + (half - 1)) * _L + lane])
                    for u in range(_GPC)]
                poss = [jnp.where(vs[u] < tvs[u], poss[u] + half, poss[u])
                        for u in range(_GPC)]
            for u in range(_GPC):
                tv, pos = tvs[u], poss[u]
                v = plsc.load_gather(cum_v, [cbase[u] + pos * _L + lane])
                idx = pos + jnp.where(v < tv, 1, 0)
                idxc = jnp.minimum(idx, N - 1)
                j = jnp.maximum(idxc - 1, 0)
                cs_j = plsc.load_gather(cum_v, [cbase[u] + j * _L + lane])
                th_next = plsc.load_gather(buf, [rows[u], idxc])
                s1 = (tv - cs_j) / th_next
                jnf = idxc.astype(jnp.float32)
                jf = j.astype(jnp.float32)
                x_cand = jnf - _EPS + 2.0 * _EPS * s1
                x = jnp.where(jnp.logical_and(s1 == 0.0, j > 0),
                              jf - 1.0 + _EPS, x_cand)
                x_v[pl.ds((ci * _GPC + u) * _L, _L)] = x

        def chunkpair(cp, _):
            for par in range(2):
                ci = 2 * cp + par
                pltpu.make_async_copy(chunk_src(0), chunk_dst(bufs[par]),
                                      sems[par]).wait()
                nci = ci + 1

                @pl.when(nci < nchunks)
                def _():
                    pltpu.async_copy(chunk_src(nci), chunk_dst(bufs[1 - par]),
                                     sems[1 - par])

                process_chunk(bufs[par], ci)
            return 0

        lax.fori_loop(0, nchunks // 2, chunkpair, 0)
        pltpu.sync_copy(x_v, out_hbm.at[pl.ds(row0, rows_per_w)])

    return sc_call


@jax.jit
def kernel(theta, t, Y_train):
    B, N = theta.shape
    del Y_train
    return _make_sc_call(B, N)(theta, t)
